# baseline (speedup 1.0000x reference)
import functools

import jax
import jax.numpy as jnp
from jax import lax
from jax.experimental import pallas as pl
from jax.experimental.pallas import tpu as pltpu
from jax.experimental.pallas import tpu_sc as plsc

N_NODES = 10000
N_PAD = 10240
D_FEAT = 128
N_EDGES = 320000

NC, NS, L = 2, 16, 16
NW = NC * NS
CHUNK = 80
N_CHUNKS = 127
E_PER_W = N_CHUNKS * CHUNK
E_TOT = NW * E_PER_W
ROWS_PER_TILE = N_PAD // NS


def _sc_aggregate(x, src3, dst1, w1):
    mesh = plsc.VectorSubcoreMesh(core_axis_name="c", subcore_axis_name="s")

    @functools.partial(
        pl.kernel,
        out_type=jax.ShapeDtypeStruct((NC * N_PAD, D_FEAT), jnp.float32),
        mesh=mesh,
        scratch_types=[
            pltpu.VMEM((N_CHUNKS, CHUNK), jnp.int32),
            [pltpu.VMEM((CHUNK, D_FEAT), jnp.float32)] * 2,
            [pltpu.VMEM((CHUNK,), jnp.int32)] * 2,
            [pltpu.VMEM((CHUNK,), jnp.float32)] * 2,
            pltpu.VMEM_SHARED((N_PAD, D_FEAT), jnp.float32),
            [pltpu.SemaphoreType.DMA] * 2,
        ],
    )
    def k(x_hbm, src_hbm, dst_hbm, w_hbm, out_hbm,
          src_v, rows, dsts, ws, acc_sh, gsem):
        c = lax.axis_index("c")
        s = lax.axis_index("s")
        wid = c * NS + s

        pltpu.sync_copy(src_hbm.at[wid], src_v)

        zero16 = jnp.zeros((L,), jnp.float32)

        def zrow(r, _):
            for cc in range(D_FEAT // L):
                rows[0][r, pl.ds(cc * L, L)] = zero16
            return 0

        lax.fori_loop(0, CHUNK, zrow, 0)
        for j in range(ROWS_PER_TILE // CHUNK):
            pltpu.sync_copy(
                rows[0],
                acc_sh.at[pl.ds(s * ROWS_PER_TILE + j * CHUNK, CHUNK), :])
        tail = ROWS_PER_TILE % CHUNK
        if tail:
            pltpu.sync_copy(
                rows[0].at[pl.ds(0, tail), :],
                acc_sh.at[pl.ds(s * ROWS_PER_TILE + ROWS_PER_TILE - tail,
                                tail), :])
        plsc.subcore_barrier()

        def fire(i, b):
            base = wid * E_PER_W + i * CHUNK
            pltpu.async_copy(x_hbm.at[src_v.at[i]], rows[b], gsem[b])
            pltpu.async_copy(dst_hbm.at[pl.ds(base, CHUNK)], dsts[b], gsem[b])
            pltpu.async_copy(w_hbm.at[pl.ds(base, CHUNK)], ws[b], gsem[b])

        def drain(i, b):
            base = wid * E_PER_W + i * CHUNK
            pltpu.make_async_copy(
                x_hbm.at[src_v.at[i]], rows[b], gsem[b]).wait()
            pltpu.make_async_copy(
                dst_hbm.at[pl.ds(base, CHUNK)], dsts[b], gsem[b]).wait()
            pltpu.make_async_copy(
                w_hbm.at[pl.ds(base, CHUNK)], ws[b], gsem[b]).wait()

        def scale_scatter(b):
            def g_body(g, _):
                wvec = ws[b][pl.ds(g * L, L)]
                for e16 in range(L):
                    wv = jnp.full((L,), wvec[e16])
                    e = g * L + e16
                    for cc in range(D_FEAT // L):
                        sl = pl.ds(cc * L, L)
                        rows[b][e, sl] = rows[b][e, sl] * wv
                return 0

            lax.fori_loop(0, CHUNK // L, g_body, 0)
            pltpu.sync_copy(rows[b], acc_sh.at[dsts[b]], add=True)

        fire(0, 0)

        def pair_body(j, _):
            a = 2 * j
            fire(a + 1, 1)
            drain(a, 0)
            scale_scatter(0)
            fire(a + 2, 0)
            drain(a + 1, 1)
            scale_scatter(1)
            return 0

        lax.fori_loop(0, (N_CHUNKS - 1) // 2, pair_body, 0)
        drain(N_CHUNKS - 1, 0)
        scale_scatter(0)
        plsc.subcore_barrier()

        row0 = c * N_PAD + s * ROWS_PER_TILE
        pltpu.sync_copy(acc_sh.at[pl.ds(s * ROWS_PER_TILE, ROWS_PER_TILE), :],
                        out_hbm.at[pl.ds(row0, ROWS_PER_TILE), :])

    return k(x, src3, dst1, w1)


def _tc_finish(agg, wt):
    BLK = 1024
    assert N_PAD % BLK == 0

    def body(a0_ref, a1_ref, wt_ref, o_ref):
        sacc = a0_ref[...] + a1_ref[...]
        h = jnp.dot(sacc, wt_ref[...], preferred_element_type=jnp.float32)
        o_ref[...] = 0.5 * h * (1.0 + lax.erf(h * 0.7071067811865476))

    return pl.pallas_call(
        body,
        grid=(N_PAD // BLK,),
        in_specs=[
            pl.BlockSpec((BLK, D_FEAT), lambda i: (i, 0)),
            pl.BlockSpec((BLK, D_FEAT),
                         lambda i: (N_PAD // BLK + i, 0)),
            pl.BlockSpec((D_FEAT, D_FEAT), lambda i: (0, 0)),
        ],
        out_specs=pl.BlockSpec((BLK, D_FEAT), lambda i: (i, 0)),
        out_shape=jax.ShapeDtypeStruct((N_NODES, D_FEAT), jnp.float32),
    )(agg, agg, wt)


def kernel(x, edge_index, edge_weight, W):
    npad = E_TOT - N_EDGES
    src1 = jnp.concatenate([edge_index[1], jnp.zeros((npad,), jnp.int32)])
    dst1 = jnp.concatenate([edge_index[0], jnp.zeros((npad,), jnp.int32)])
    w1 = jnp.concatenate([edge_weight, jnp.zeros((npad,), jnp.float32)])
    src3 = src1.reshape(NW, N_CHUNKS, CHUNK)
    agg = _sc_aggregate(x, src3, dst1, w1)
    return _tc_finish(agg, W.T)


# speedup vs baseline: 2.6660x; 2.6660x over previous
import functools

import jax
import jax.numpy as jnp
from jax import lax
from jax.experimental import pallas as pl
from jax.experimental.pallas import tpu as pltpu
from jax.experimental.pallas import tpu_sc as plsc

N_NODES = 10000
N_PAD = 10240
D_FEAT = 128
N_EDGES = 320000

NC, NS, L = 2, 16, 16
NW = NC * NS
CHUNK = 80
N_CHUNKS = 126
E_PER_W = N_CHUNKS * CHUNK
E_TOT = NW * E_PER_W
ROWS_PER_TILE = N_PAD // NS


def _sc_aggregate(x, src3, dst1, w1):
    mesh = plsc.VectorSubcoreMesh(core_axis_name="c", subcore_axis_name="s")

    @functools.partial(
        pl.kernel,
        out_type=jax.ShapeDtypeStruct((NC * N_PAD, D_FEAT), jnp.float32),
        mesh=mesh,
        scratch_types=[
            pltpu.VMEM((N_CHUNKS, CHUNK), jnp.int32),
            [pltpu.VMEM((CHUNK, D_FEAT), jnp.float32)] * 3,
            [pltpu.VMEM((CHUNK,), jnp.int32)] * 3,
            [pltpu.VMEM((CHUNK,), jnp.float32)] * 3,
            pltpu.VMEM_SHARED((N_PAD, D_FEAT), jnp.float32),
            [pltpu.SemaphoreType.DMA] * 3,
            [pltpu.SemaphoreType.DMA] * 3,
        ],
    )
    def k(x_hbm, src_hbm, dst_hbm, w_hbm, out_hbm,
          src_v, rows, dsts, ws, acc_sh, gsem, ssem):
        c = lax.axis_index("c")
        s = lax.axis_index("s")
        wid = c * NS + s

        pltpu.sync_copy(src_hbm.at[wid], src_v)

        zero16 = jnp.zeros((L,), jnp.float32)

        def zrow(r, _):
            for cc in range(D_FEAT // L):
                rows[0][r, pl.ds(cc * L, L)] = zero16
            return 0

        lax.fori_loop(0, CHUNK, zrow, 0)
        for j in range(ROWS_PER_TILE // CHUNK):
            pltpu.sync_copy(
                rows[0],
                acc_sh.at[pl.ds(s * ROWS_PER_TILE + j * CHUNK, CHUNK), :])
        plsc.subcore_barrier()

        def fire(i, b):
            base = wid * E_PER_W + i * CHUNK
            pltpu.async_copy(x_hbm.at[src_v.at[i]], rows[b], gsem[b])
            pltpu.async_copy(dst_hbm.at[pl.ds(base, CHUNK)], dsts[b], gsem[b])
            pltpu.async_copy(w_hbm.at[pl.ds(base, CHUNK)], ws[b], gsem[b])

        def drain(i, b):
            base = wid * E_PER_W + i * CHUNK
            pltpu.make_async_copy(
                x_hbm.at[src_v.at[i]], rows[b], gsem[b]).wait()
            pltpu.make_async_copy(
                dst_hbm.at[pl.ds(base, CHUNK)], dsts[b], gsem[b]).wait()
            pltpu.make_async_copy(
                w_hbm.at[pl.ds(base, CHUNK)], ws[b], gsem[b]).wait()

        def scale(b):
            def g_body(g, _):
                wvec = ws[b][pl.ds(g * L, L)]
                for e16 in range(L):
                    wv = jnp.full((L,), wvec[e16])
                    e = g * L + e16
                    for cc in range(D_FEAT // L):
                        sl = pl.ds(cc * L, L)
                        rows[b][e, sl] = rows[b][e, sl] * wv
                return 0

            lax.fori_loop(0, CHUNK // L, g_body, 0)

        def fire_scatter(b):
            pltpu.async_copy(rows[b], acc_sh.at[dsts[b]], ssem[b], add=True)

        def wait_scatter(b):
            pltpu.make_async_copy(rows[b], acc_sh.at[dsts[b]], ssem[b]).wait()

        fire(0, 0)
        fire(1, 1)
        drain(0, 0)
        scale(0)
        fire_scatter(0)
        fire(2, 2)

        def tri_body(j, _):
            for t in range(3):
                ci = 3 * j + 1 + t
                b = (1 + t) % 3
                bp = (b + 2) % 3
                drain(ci, b)
                scale(b)
                wait_scatter(bp)
                fire_scatter(b)
                fire(ci + 2, bp)
            return 0

        lax.fori_loop(0, (N_CHUNKS - 3) // 3, tri_body, 0)
        drain(N_CHUNKS - 2, 1)
        scale(1)
        wait_scatter(0)
        fire_scatter(1)
        drain(N_CHUNKS - 1, 2)
        scale(2)
        wait_scatter(1)
        fire_scatter(2)
        wait_scatter(2)
        plsc.subcore_barrier()

        row0 = c * N_PAD + s * ROWS_PER_TILE
        pltpu.sync_copy(acc_sh.at[pl.ds(s * ROWS_PER_TILE, ROWS_PER_TILE), :],
                        out_hbm.at[pl.ds(row0, ROWS_PER_TILE), :])

    return k(x, src3, dst1, w1)


def _tc_finish(agg, wt):
    BLK = 1024
    assert N_PAD % BLK == 0

    def body(a0_ref, a1_ref, wt_ref, o_ref):
        sacc = a0_ref[...] + a1_ref[...]
        h = jnp.dot(sacc, wt_ref[...], preferred_element_type=jnp.float32)
        o_ref[...] = 0.5 * h * (1.0 + lax.erf(h * 0.7071067811865476))

    return pl.pallas_call(
        body,
        grid=(N_PAD // BLK,),
        in_specs=[
            pl.BlockSpec((BLK, D_FEAT), lambda i: (i, 0)),
            pl.BlockSpec((BLK, D_FEAT),
                         lambda i: (N_PAD // BLK + i, 0)),
            pl.BlockSpec((D_FEAT, D_FEAT), lambda i: (0, 0)),
        ],
        out_specs=pl.BlockSpec((BLK, D_FEAT), lambda i: (i, 0)),
        out_shape=jax.ShapeDtypeStruct((N_NODES, D_FEAT), jnp.float32),
    )(agg, agg, wt)


def kernel(x, edge_index, edge_weight, W):
    npad = E_TOT - N_EDGES
    spread = jnp.arange(npad, dtype=jnp.int32) % N_NODES
    src1 = jnp.concatenate([edge_index[1], spread])
    dst1 = jnp.concatenate([edge_index[0], spread])
    w1 = jnp.concatenate([edge_weight, jnp.zeros((npad,), jnp.float32)])
    src3 = src1.reshape(NW, N_CHUNKS, CHUNK)
    agg = _sc_aggregate(x, src3, dst1, w1)
    return _tc_finish(agg, W.T)
